# baseline (speedup 1.0000x reference)
import functools
import jax
import jax.numpy as jnp
from jax import lax
from jax.experimental import pallas as pl
from jax.experimental.pallas import tpu as pltpu
import jax.experimental.pallas.tpu_sc as plsc

_LANES = 16
_CHUNK_ELEMS = 32768


def kernel(x, pos_table):
    batch, ctx, dim = x.shape
    n_elems = batch * ctx * dim
    pos_elems = ctx * dim
    xf = x.reshape(n_elems)
    posf = pos_table.reshape(pos_elems)

    num_workers = 32
    elems_per_w = n_elems // num_workers
    n_chunks = elems_per_w // _CHUNK_ELEMS
    mesh = plsc.VectorSubcoreMesh(core_axis_name="c", subcore_axis_name="s")

    @functools.partial(
        pl.kernel,
        out_type=jax.ShapeDtypeStruct((n_elems,), jnp.float32),
        mesh=mesh,
        scratch_types=[
            pltpu.VMEM((_CHUNK_ELEMS,), jnp.float32),
            pltpu.VMEM((_CHUNK_ELEMS,), jnp.float32),
        ],
    )
    def sc_add(x_hbm, pos_hbm, out_hbm, xbuf, pbuf):
        wid = lax.axis_index("s") * 2 + lax.axis_index("c")
        base_w = wid * elems_per_w
        pos_base_w = base_w % pos_elems

        def chunk_body(k, carry):
            base = base_w + k * _CHUNK_ELEMS
            pos_base = pos_base_w + k * _CHUNK_ELEMS
            pltpu.sync_copy(x_hbm.at[pl.ds(base, _CHUNK_ELEMS)], xbuf)
            pltpu.sync_copy(pos_hbm.at[pl.ds(pos_base, _CHUNK_ELEMS)], pbuf)

            @plsc.parallel_loop(0, _CHUNK_ELEMS, _LANES, unroll=8)
            def add_body(i):
                xbuf[pl.ds(i, _LANES)] = (
                    xbuf[pl.ds(i, _LANES)] + pbuf[pl.ds(i, _LANES)]
                )

            pltpu.sync_copy(xbuf, out_hbm.at[pl.ds(base, _CHUNK_ELEMS)])
            return carry

        lax.fori_loop(0, n_chunks, chunk_body, 0)

    return sc_add(xf, posf).reshape(batch, ctx, dim)


# speedup vs baseline: 1.0831x; 1.0831x over previous
import functools
import jax
import jax.numpy as jnp
from jax import lax
from jax.experimental import pallas as pl
from jax.experimental.pallas import tpu as pltpu
import jax.experimental.pallas.tpu_sc as plsc

_LANES = 16
_CHUNK_ELEMS = 16384
_NBUF = 2


def kernel(x, pos_table):
    batch, ctx, dim = x.shape
    n_elems = batch * ctx * dim
    pos_elems = ctx * dim
    xf = x.reshape(n_elems)
    posf = pos_table.reshape(pos_elems)

    num_workers = 32
    elems_per_w = n_elems // num_workers
    n_chunks = elems_per_w // _CHUNK_ELEMS
    mesh = plsc.VectorSubcoreMesh(core_axis_name="c", subcore_axis_name="s")

    @functools.partial(
        pl.kernel,
        out_type=jax.ShapeDtypeStruct((n_elems,), jnp.float32),
        mesh=mesh,
        scratch_types=[
            pltpu.VMEM((_NBUF, _CHUNK_ELEMS), jnp.float32),
            pltpu.VMEM((_NBUF, _CHUNK_ELEMS), jnp.float32),
            pltpu.VMEM((_NBUF, _CHUNK_ELEMS), jnp.float32),
            [pltpu.SemaphoreType.DMA] * _NBUF,
            [pltpu.SemaphoreType.DMA] * _NBUF,
        ],
    )
    def sc_add(x_hbm, pos_hbm, out_hbm, xb, pb, ob, ld_sems, st_sems):
        wid = lax.axis_index("s") * 2 + lax.axis_index("c")
        base_w = wid * elems_per_w
        pos_base_w = base_w % pos_elems

        def start_loads(k):
            s = k % _NBUF
            base = base_w + k * _CHUNK_ELEMS
            pos_base = pos_base_w + k * _CHUNK_ELEMS
            ld_x = pltpu.async_copy(
                x_hbm.at[pl.ds(base, _CHUNK_ELEMS)], xb.at[s], ld_sems[s]
            )
            ld_p = pltpu.async_copy(
                pos_hbm.at[pl.ds(pos_base, _CHUNK_ELEMS)], pb.at[s], ld_sems[s]
            )
            return ld_x, ld_p

        pending_loads = [start_loads(k) for k in range(min(_NBUF, n_chunks))]
        pending_store = [None] * _NBUF

        for k in range(n_chunks):
            s = k % _NBUF
            ld_x, ld_p = pending_loads.pop(0)
            ld_x.wait()
            ld_p.wait()
            if pending_store[s] is not None:
                pending_store[s].wait()
                pending_store[s] = None

            @plsc.parallel_loop(0, _CHUNK_ELEMS, _LANES, unroll=8)
            def add_body(i):
                ob[s, pl.ds(i, _LANES)] = (
                    xb[s, pl.ds(i, _LANES)] + pb[s, pl.ds(i, _LANES)]
                )

            nxt = k + _NBUF
            if nxt < n_chunks:
                pending_loads.append(start_loads(nxt))
            base = base_w + k * _CHUNK_ELEMS
            pending_store[s] = pltpu.async_copy(
                ob.at[s], out_hbm.at[pl.ds(base, _CHUNK_ELEMS)], st_sems[s]
            )

        for st in pending_store:
            if st is not None:
                st.wait()

    return sc_add(xf, posf).reshape(batch, ctx, dim)


# speedup vs baseline: 3.8402x; 3.5455x over previous
import functools
import jax
import jax.numpy as jnp
from jax import lax
from jax.experimental import pallas as pl
from jax.experimental.pallas import tpu as pltpu
import jax.experimental.pallas.tpu_sc as plsc

_LANES = 16
_CHUNK_ROWS = 16
_NBUF = 2
_NUM_WORKERS = 32


def kernel(x, pos_table):
    batch, ctx, dim = x.shape
    rows_per_w = ctx // _NUM_WORKERS
    n_chunks = rows_per_w // _CHUNK_ROWS
    n_steps = n_chunks * batch
    mesh = plsc.VectorSubcoreMesh(core_axis_name="c", subcore_axis_name="s")

    @functools.partial(
        pl.kernel,
        out_type=jax.ShapeDtypeStruct(x.shape, x.dtype),
        mesh=mesh,
        scratch_types=[
            pltpu.VMEM((_NBUF, _CHUNK_ROWS, dim), jnp.float32),
            pltpu.VMEM((_NBUF, _CHUNK_ROWS, dim), jnp.float32),
            pltpu.VMEM((_NBUF, _CHUNK_ROWS, dim), jnp.float32),
            [pltpu.SemaphoreType.DMA] * _NBUF,
            [pltpu.SemaphoreType.DMA] * _NBUF,
            [pltpu.SemaphoreType.DMA] * _NBUF,
        ],
        compiler_params=pltpu.CompilerParams(use_tc_tiling_on_sc=True),
    )
    def sc_add(x_hbm, pos_hbm, out_hbm, xb, pb, ob, lx_sems, lp_sems, st_sems):
        wid = lax.axis_index("s") * 2 + lax.axis_index("c")
        row0_w = wid * rows_per_w

        def start_x(t):
            k, b = divmod(t, batch)
            r0 = row0_w + k * _CHUNK_ROWS
            return pltpu.async_copy(
                x_hbm.at[b, pl.ds(r0, _CHUNK_ROWS), :],
                xb.at[t % _NBUF],
                lx_sems[t % _NBUF],
            )

        def start_p(k):
            r0 = row0_w + k * _CHUNK_ROWS
            return pltpu.async_copy(
                pos_hbm.at[pl.ds(r0, _CHUNK_ROWS), :],
                pb.at[k % _NBUF],
                lp_sems[k % _NBUF],
            )

        ld_x = {t: start_x(t) for t in range(min(_NBUF, n_steps))}
        ld_p = {k: start_p(k) for k in range(min(_NBUF, n_chunks))}
        st = [None] * _NBUF

        for t in range(n_steps):
            k, b = divmod(t, batch)
            s = t % _NBUF
            ld_x.pop(t).wait()
            if b == 0:
                ld_p.pop(k).wait()
            if st[s] is not None:
                st[s].wait()
                st[s] = None

            @plsc.parallel_loop(0, _CHUNK_ROWS * dim, _LANES, unroll=8)
            def add_body(i):
                r = i // dim
                c = i % dim
                ob[s, r, pl.ds(c, _LANES)] = (
                    xb[s, r, pl.ds(c, _LANES)] + pb[k % _NBUF, r, pl.ds(c, _LANES)]
                )

            if t + _NBUF < n_steps:
                ld_x[t + _NBUF] = start_x(t + _NBUF)
            if b == batch - 1 and k + _NBUF < n_chunks:
                ld_p[k + _NBUF] = start_p(k + _NBUF)
            r0 = row0_w + k * _CHUNK_ROWS
            st[s] = pltpu.async_copy(
                ob.at[s],
                out_hbm.at[b, pl.ds(r0, _CHUNK_ROWS), :],
                st_sems[s],
            )

        for d in st:
            if d is not None:
                d.wait()

    return sc_add(x, pos_table)


# speedup vs baseline: 3.8604x; 1.0053x over previous
import functools
import jax
import jax.numpy as jnp
from jax import lax
from jax.experimental import pallas as pl
from jax.experimental.pallas import tpu as pltpu
import jax.experimental.pallas.tpu_sc as plsc

_LANES = 16
_CHUNK_ROWS = 16
_NBUF = 4
_PBUF = 2
_NUM_WORKERS = 32


def kernel(x, pos_table):
    batch, ctx, dim = x.shape
    rows_per_w = ctx // _NUM_WORKERS
    n_chunks = rows_per_w // _CHUNK_ROWS
    n_steps = n_chunks * batch
    mesh = plsc.VectorSubcoreMesh(core_axis_name="c", subcore_axis_name="s")

    @functools.partial(
        pl.kernel,
        out_type=jax.ShapeDtypeStruct(x.shape, x.dtype),
        mesh=mesh,
        scratch_types=[
            pltpu.VMEM((_NBUF, _CHUNK_ROWS, dim), jnp.float32),
            pltpu.VMEM((_PBUF, _CHUNK_ROWS, dim), jnp.float32),
            [pltpu.SemaphoreType.DMA] * _NBUF,
            [pltpu.SemaphoreType.DMA] * _PBUF,
            [pltpu.SemaphoreType.DMA] * _NBUF,
        ],
        compiler_params=pltpu.CompilerParams(use_tc_tiling_on_sc=True),
    )
    def sc_add(x_hbm, pos_hbm, out_hbm, ob, pb, lx_sems, lp_sems, st_sems):
        wid = lax.axis_index("s") * 2 + lax.axis_index("c")
        row0_w = wid * rows_per_w

        def start_x(t):
            k, b = divmod(t, batch)
            r0 = row0_w + k * _CHUNK_ROWS
            return pltpu.async_copy(
                x_hbm.at[b, pl.ds(r0, _CHUNK_ROWS), :],
                ob.at[t % _NBUF],
                lx_sems[t % _NBUF],
            )

        def start_p(k):
            r0 = row0_w + k * _CHUNK_ROWS
            return pltpu.async_copy(
                pos_hbm.at[pl.ds(r0, _CHUNK_ROWS), :],
                pb.at[k % _PBUF],
                lp_sems[k % _PBUF],
            )

        ld_x = {t: start_x(t) for t in range(min(2, n_steps))}
        ld_p = {k: start_p(k) for k in range(min(_PBUF, n_chunks))}
        st = [None] * _NBUF

        for t in range(n_steps):
            k, b = divmod(t, batch)
            s = t % _NBUF
            ld_x.pop(t).wait()
            if b == 0:
                ld_p.pop(k).wait()

            @plsc.parallel_loop(0, _CHUNK_ROWS * dim, _LANES, unroll=8)
            def add_body(i):
                r = i // dim
                c = i % dim
                plsc.addupdate(
                    ob.at[s, r, pl.ds(c, _LANES)],
                    pb[k % _PBUF, r, pl.ds(c, _LANES)],
                )

            r0 = row0_w + k * _CHUNK_ROWS
            st[s] = pltpu.async_copy(
                ob.at[s],
                out_hbm.at[b, pl.ds(r0, _CHUNK_ROWS), :],
                st_sems[s],
            )
            nxt = t + 2
            if nxt < n_steps:
                s2 = nxt % _NBUF
                if st[s2] is not None:
                    st[s2].wait()
                    st[s2] = None
                ld_x[nxt] = start_x(nxt)
            if b == batch - 1 and k + _PBUF < n_chunks:
                ld_p[k + _PBUF] = start_p(k + _PBUF)

        for d in st:
            if d is not None:
                d.wait()

    return sc_add(x, pos_table)


# speedup vs baseline: 4.0683x; 1.0539x over previous
import functools
import jax
import jax.numpy as jnp
from jax import lax
from jax.experimental import pallas as pl
from jax.experimental.pallas import tpu as pltpu
import jax.experimental.pallas.tpu_sc as plsc

_LANES = 16
_CHUNK_ROWS = 16
_NBUF = 4
_PBUF = 2
_NUM_WORKERS = 32


def kernel(x, pos_table):
    batch, ctx, dim = x.shape
    rows_per_w = ctx // _NUM_WORKERS
    n_chunks = rows_per_w // _CHUNK_ROWS
    n_steps = n_chunks * batch
    mesh = plsc.VectorSubcoreMesh(core_axis_name="c", subcore_axis_name="s")

    @functools.partial(
        pl.kernel,
        out_type=jax.ShapeDtypeStruct(x.shape, x.dtype),
        mesh=mesh,
        scratch_types=[
            pltpu.VMEM((_NBUF, _CHUNK_ROWS, dim), jnp.float32),
            pltpu.VMEM((_PBUF, _CHUNK_ROWS, dim), jnp.float32),
            [pltpu.SemaphoreType.DMA] * _NBUF,
            [pltpu.SemaphoreType.DMA] * _PBUF,
            [pltpu.SemaphoreType.DMA] * _NBUF,
        ],
        compiler_params=pltpu.CompilerParams(use_tc_tiling_on_sc=True),
    )
    def sc_add(x_hbm, pos_hbm, out_hbm, ob, pb, lx_sems, lp_sems, st_sems):
        wid = lax.axis_index("s") * 2 + lax.axis_index("c")
        row0_w = wid * rows_per_w

        def start_x(t):
            k, b = divmod(t, batch)
            r0 = row0_w + k * _CHUNK_ROWS
            return pltpu.async_copy(
                x_hbm.at[b, pl.ds(r0, _CHUNK_ROWS), :],
                ob.at[t % _NBUF],
                lx_sems[t % _NBUF],
            )

        def start_p(k):
            r0 = row0_w + k * _CHUNK_ROWS
            return pltpu.async_copy(
                pos_hbm.at[pl.ds(r0, _CHUNK_ROWS), :],
                pb.at[k % _PBUF],
                lp_sems[k % _PBUF],
            )

        ld_x = {t: start_x(t) for t in range(min(3, n_steps))}
        ld_p = {k: start_p(k) for k in range(min(_PBUF, n_chunks))}
        st = [None] * _NBUF

        for t in range(n_steps):
            k, b = divmod(t, batch)
            s = t % _NBUF
            ld_x.pop(t).wait()
            if b == 0:
                ld_p.pop(k).wait()

            @plsc.parallel_loop(0, _CHUNK_ROWS * dim, _LANES, unroll=8)
            def add_body(i):
                r = i // dim
                c = i % dim
                plsc.addupdate(
                    ob.at[s, r, pl.ds(c, _LANES)],
                    pb[k % _PBUF, r, pl.ds(c, _LANES)],
                )

            r0 = row0_w + k * _CHUNK_ROWS
            st[s] = pltpu.async_copy(
                ob.at[s],
                out_hbm.at[b, pl.ds(r0, _CHUNK_ROWS), :],
                st_sems[s],
            )
            nxt = t + 3
            if nxt < n_steps:
                s2 = nxt % _NBUF
                if st[s2] is not None:
                    st[s2].wait()
                    st[s2] = None
                ld_x[nxt] = start_x(nxt)
            if b == batch - 1 and k + _PBUF < n_chunks:
                ld_p[k + _PBUF] = start_p(k + _PBUF)

        for d in st:
            if d is not None:
                d.wait()

    return sc_add(x, pos_table)


# speedup vs baseline: 4.0730x; 1.0011x over previous
import functools
import jax
import jax.numpy as jnp
from jax import lax
from jax.experimental import pallas as pl
from jax.experimental.pallas import tpu as pltpu
import jax.experimental.pallas.tpu_sc as plsc

_LANES = 16
_CHUNK_ROWS = 16
_NBUF = 5
_PBUF = 2
_NUM_WORKERS = 32


def kernel(x, pos_table):
    batch, ctx, dim = x.shape
    rows_per_w = ctx // _NUM_WORKERS
    n_chunks = rows_per_w // _CHUNK_ROWS
    n_steps = n_chunks * batch
    mesh = plsc.VectorSubcoreMesh(core_axis_name="c", subcore_axis_name="s")

    @functools.partial(
        pl.kernel,
        out_type=jax.ShapeDtypeStruct(x.shape, x.dtype),
        mesh=mesh,
        scratch_types=[
            pltpu.VMEM((_NBUF, _CHUNK_ROWS, dim), jnp.float32),
            pltpu.VMEM((_PBUF, _CHUNK_ROWS, dim), jnp.float32),
            [pltpu.SemaphoreType.DMA] * _NBUF,
            [pltpu.SemaphoreType.DMA] * _PBUF,
            [pltpu.SemaphoreType.DMA] * _NBUF,
        ],
        compiler_params=pltpu.CompilerParams(use_tc_tiling_on_sc=True),
    )
    def sc_add(x_hbm, pos_hbm, out_hbm, ob, pb, lx_sems, lp_sems, st_sems):
        wid = lax.axis_index("s") * 2 + lax.axis_index("c")
        row0_w = wid * rows_per_w

        def start_x(t):
            k, b = divmod(t, batch)
            r0 = row0_w + k * _CHUNK_ROWS
            return pltpu.async_copy(
                x_hbm.at[b, pl.ds(r0, _CHUNK_ROWS), :],
                ob.at[t % _NBUF],
                lx_sems[t % _NBUF],
            )

        def start_p(k):
            r0 = row0_w + k * _CHUNK_ROWS
            return pltpu.async_copy(
                pos_hbm.at[pl.ds(r0, _CHUNK_ROWS), :],
                pb.at[k % _PBUF],
                lp_sems[k % _PBUF],
            )

        ld_x = {t: start_x(t) for t in range(min(4, n_steps))}
        ld_p = {k: start_p(k) for k in range(min(_PBUF, n_chunks))}
        st = [None] * _NBUF

        for t in range(n_steps):
            k, b = divmod(t, batch)
            s = t % _NBUF
            ld_x.pop(t).wait()
            if b == 0:
                ld_p.pop(k).wait()

            @plsc.parallel_loop(0, _CHUNK_ROWS * dim, _LANES, unroll=8)
            def add_body(i):
                r = i // dim
                c = i % dim
                plsc.addupdate(
                    ob.at[s, r, pl.ds(c, _LANES)],
                    pb[k % _PBUF, r, pl.ds(c, _LANES)],
                )

            r0 = row0_w + k * _CHUNK_ROWS
            st[s] = pltpu.async_copy(
                ob.at[s],
                out_hbm.at[b, pl.ds(r0, _CHUNK_ROWS), :],
                st_sems[s],
            )
            nxt = t + 4
            if nxt < n_steps:
                s2 = nxt % _NBUF
                if st[s2] is not None:
                    st[s2].wait()
                    st[s2] = None
                ld_x[nxt] = start_x(nxt)
            if b == batch - 1 and k + _PBUF < n_chunks:
                ld_p[k + _PBUF] = start_p(k + _PBUF)

        for d in st:
            if d is not None:
                d.wait()

    return sc_add(x, pos_table)
